# Initial kernel scaffold; baseline (speedup 1.0000x reference)
import jax
import jax.numpy as jnp
from jax.experimental import pallas as pl


def kernel(x, edge_index, W, b, noise):
    raise NotImplementedError("write your pallas kernel here")


# speedup vs baseline: 5.7132x; 5.7132x over previous
import functools

import jax
import jax.numpy as jnp
from jax import lax
from jax.experimental import pallas as pl
from jax.experimental.pallas import tpu as pltpu
from jax.experimental.pallas import tpu_sc as plsc

N = 10000
E = 320000
D = 128
SIGMA = 0.1

NC = 2
NS = 16
NW = NC * NS
CHUNK = 128
EPAD = ((E + NW * CHUNK - 1) // (NW * CHUNK)) * (NW * CHUNK)
NCHUNK = EPAD // (NW * CHUNK)
NPAD = ((N + NS * 8) // (NS * 8)) * (NS * 8)
ROWS_PER_TILE = NPAD // NS


def _normalize_body(x_ref, o_ref):
    x = x_ref[...]
    s = jnp.sum(x * x, axis=1, keepdims=True)
    o_ref[...] = x * lax.rsqrt(jnp.maximum(s, 1e-24))


def _normalize(x):
    return pl.pallas_call(
        _normalize_body,
        out_shape=jax.ShapeDtypeStruct((N, D), jnp.float32),
    )(x)


_sc_mesh = plsc.VectorSubcoreMesh(core_axis_name="c", subcore_axis_name="s")


@functools.partial(
    pl.kernel,
    out_type=(
        jax.ShapeDtypeStruct((NC, NPAD, D), jnp.float32),
        jax.ShapeDtypeStruct((NW, NPAD), jnp.float32),
    ),
    mesh=_sc_mesh,
    scratch_types=[
        pltpu.VMEM((NCHUNK, CHUNK), jnp.int32),
        pltpu.VMEM((NCHUNK, CHUNK), jnp.int32),
        pltpu.VMEM((CHUNK, D), jnp.float32),
        pltpu.VMEM((NPAD,), jnp.float32),
        pltpu.VMEM_SHARED((NPAD, D), jnp.float32),
        pltpu.SemaphoreType.DMA,
    ],
    compiler_params=pltpu.CompilerParams(needs_layout_passes=False),
)
def _sc_scatter(xn_hbm, src_hbm, dst_hbm, zrow_hbm, zdeg_hbm,
                agg_out, deg_out, src_v, dst_v, rows_v, deg_v, agg_sh, sem):
    c = lax.axis_index("c")
    s = lax.axis_index("s")
    g = c * NS + s

    pltpu.sync_copy(src_hbm.at[g], src_v)
    pltpu.sync_copy(dst_hbm.at[g], dst_v)
    pltpu.sync_copy(zrow_hbm, agg_sh.at[pl.ds(s * ROWS_PER_TILE, ROWS_PER_TILE)])
    pltpu.sync_copy(zdeg_hbm, deg_v)
    plsc.subcore_barrier()

    ones = jnp.full((16,), 1.0, dtype=jnp.float32)

    def body(j, carry):
        pltpu.async_copy(xn_hbm.at[src_v.at[j]], rows_v, sem).wait()
        pltpu.sync_copy(rows_v, agg_sh.at[dst_v.at[j]], add=True)
        for i in range(CHUNK // 16):
            idx16 = dst_v[j, pl.ds(i * 16, 16)]
            plsc.addupdate_scatter(deg_v, [idx16], ones)
        return carry

    lax.fori_loop(0, NCHUNK, body, 0)
    plsc.subcore_barrier()

    pltpu.sync_copy(
        agg_sh.at[pl.ds(s * ROWS_PER_TILE, ROWS_PER_TILE)],
        agg_out.at[c, pl.ds(s * ROWS_PER_TILE, ROWS_PER_TILE)],
    )
    pltpu.sync_copy(deg_v, deg_out.at[g])


_FIN_BLK = 1000


def _final_body(a0_ref, a1_ref, deg_ref, w_ref, b_ref, noise_ref, o_ref):
    deg = jnp.sum(deg_ref[...], axis=1)
    agg = a0_ref[...] + a1_ref[...]
    mean = agg / jnp.maximum(deg, 1.0)[:, None]
    o_ref[...] = (
        jnp.dot(mean, w_ref[...], preferred_element_type=jnp.float32)
        + b_ref[...]
        + noise_ref[...] * SIGMA
    )


def _final(a0, a1, deg_part, W, b2, noise):
    return pl.pallas_call(
        _final_body,
        grid=(N // _FIN_BLK,),
        in_specs=[
            pl.BlockSpec((_FIN_BLK, D), lambda i: (i, 0)),
            pl.BlockSpec((_FIN_BLK, D), lambda i: (i, 0)),
            pl.BlockSpec((_FIN_BLK, NW), lambda i: (i, 0)),
            pl.BlockSpec((D, D), lambda i: (0, 0)),
            pl.BlockSpec((1, D), lambda i: (0, 0)),
            pl.BlockSpec((_FIN_BLK, D), lambda i: (i, 0)),
        ],
        out_specs=pl.BlockSpec((_FIN_BLK, D), lambda i: (i, 0)),
        out_shape=jax.ShapeDtypeStruct((N, D), jnp.float32),
    )(a0, a1, deg_part, W, b2, noise)


def kernel(x, edge_index, W, b, noise):
    xn = _normalize(x)

    src = edge_index[0]
    dst = edge_index[1]
    pad = EPAD - E
    src_p = jnp.concatenate([src, jnp.zeros((pad,), jnp.int32)])
    dst_p = jnp.concatenate([dst, jnp.full((pad,), N, jnp.int32)])
    src3 = src_p.reshape(NW, NCHUNK, CHUNK)
    dst3 = dst_p.reshape(NW, NCHUNK, CHUNK)

    zrow = jnp.zeros((ROWS_PER_TILE, D), jnp.float32)
    zdeg = jnp.zeros((NPAD,), jnp.float32)

    agg_part, deg_part = _sc_scatter(xn, src3, dst3, zrow, zdeg)

    return _final(agg_part[0, :N], agg_part[1, :N], deg_part.T, W,
                  b.reshape(1, D), noise)


# speedup vs baseline: 7.5592x; 1.3231x over previous
import functools

import jax
import jax.numpy as jnp
from jax import lax
from jax.experimental import pallas as pl
from jax.experimental.pallas import tpu as pltpu
from jax.experimental.pallas import tpu_sc as plsc

N = 10000
E = 320000
D = 128
SIGMA = 0.1

NC = 2
NS = 16
NW = NC * NS
CHUNK = 96
NRING = 3
NCHUNK = -(-E // (NW * CHUNK * NRING)) * NRING
EPAD = NW * CHUNK * NCHUNK
NPAD = ((N + NS * 8) // (NS * 8)) * (NS * 8)
ROWS_PER_TILE = NPAD // NS


def _normalize_body(x_ref, o_ref):
    x = x_ref[...]
    s = jnp.sum(x * x, axis=1, keepdims=True)
    o_ref[...] = x * lax.rsqrt(jnp.maximum(s, 1e-24))


def _normalize(x):
    return pl.pallas_call(
        _normalize_body,
        out_shape=jax.ShapeDtypeStruct((N, D), jnp.float32),
    )(x)


_sc_mesh = plsc.VectorSubcoreMesh(core_axis_name="c", subcore_axis_name="s")


@functools.partial(
    pl.kernel,
    out_type=(
        jax.ShapeDtypeStruct((NC, NPAD, D), jnp.float32),
        jax.ShapeDtypeStruct((NW, NPAD), jnp.float32),
    ),
    mesh=_sc_mesh,
    scratch_types=[
        [pltpu.VMEM((2, CHUNK), jnp.int32)] * NRING,
        [pltpu.VMEM((CHUNK, D), jnp.float32)] * NRING,
        pltpu.VMEM((NPAD,), jnp.float32),
        pltpu.VMEM_SHARED((NPAD, D), jnp.float32),
        [pltpu.SemaphoreType.DMA] * NRING,
        [pltpu.SemaphoreType.DMA] * NRING,
    ],
    compiler_params=pltpu.CompilerParams(needs_layout_passes=False),
)
def _sc_scatter(xn_hbm, idx_hbm, zrow_hbm, zdeg_hbm,
                agg_out, deg_out, ib, rows, deg_v, agg_sh, isems, gsems):
    c = lax.axis_index("c")
    s = lax.axis_index("s")
    g = c * NS + s

    pltpu.sync_copy(zrow_hbm, agg_sh.at[pl.ds(s * ROWS_PER_TILE, ROWS_PER_TILE)])
    pltpu.sync_copy(zdeg_hbm, deg_v)
    plsc.subcore_barrier()

    ones = jnp.full((16,), 1.0, dtype=jnp.float32)

    def idx_fetch(j, a):
        pltpu.async_copy(idx_hbm.at[g, j], ib[a], isems[a])

    def idx_wait(j, a):
        pltpu.make_async_copy(idx_hbm.at[g, j], ib[a], isems[a]).wait()

    def row_gather(a):
        pltpu.async_copy(xn_hbm.at[ib[a].at[0]], rows[a], gsems[a])

    def row_wait(a):
        pltpu.make_async_copy(xn_hbm.at[ib[a].at[0]], rows[a], gsems[a]).wait()

    for a in range(NRING):
        idx_fetch(a, a)
    for a in range(2):
        idx_wait(a, a)
        row_gather(a)

    def body(jg, carry):
        for b in range(NRING):
            j = jg * NRING + b
            row_wait(b)
            pltpu.sync_copy(rows[b], agg_sh.at[ib[b].at[1]], add=True)
            for i in range(CHUNK // 16):
                idx16 = ib[b][1, pl.ds(i * 16, 16)]
                plsc.addupdate_scatter(deg_v, [idx16], ones)

            @pl.when(j + NRING < NCHUNK)
            def _():
                idx_fetch(j + NRING, b)

            @pl.when(j + 2 < NCHUNK)
            def _():
                b2 = (b + 2) % NRING
                idx_wait(j + 2, b2)
                row_gather(b2)

        return carry

    lax.fori_loop(0, NCHUNK // NRING, body, 0)
    plsc.subcore_barrier()

    pltpu.sync_copy(
        agg_sh.at[pl.ds(s * ROWS_PER_TILE, ROWS_PER_TILE)],
        agg_out.at[c, pl.ds(s * ROWS_PER_TILE, ROWS_PER_TILE)],
    )
    pltpu.sync_copy(deg_v, deg_out.at[g])


_FIN_BLK = 1000


def _final_body(a0_ref, a1_ref, deg_ref, w_ref, b_ref, noise_ref, o_ref):
    deg = jnp.sum(deg_ref[...], axis=1)
    agg = a0_ref[...] + a1_ref[...]
    mean = agg / jnp.maximum(deg, 1.0)[:, None]
    o_ref[...] = (
        jnp.dot(mean, w_ref[...], preferred_element_type=jnp.float32)
        + b_ref[...]
        + noise_ref[...] * SIGMA
    )


def _final(a0, a1, deg_part, W, b2, noise):
    return pl.pallas_call(
        _final_body,
        grid=(N // _FIN_BLK,),
        in_specs=[
            pl.BlockSpec((_FIN_BLK, D), lambda i: (i, 0)),
            pl.BlockSpec((_FIN_BLK, D), lambda i: (i, 0)),
            pl.BlockSpec((_FIN_BLK, NW), lambda i: (i, 0)),
            pl.BlockSpec((D, D), lambda i: (0, 0)),
            pl.BlockSpec((1, D), lambda i: (0, 0)),
            pl.BlockSpec((_FIN_BLK, D), lambda i: (i, 0)),
        ],
        out_specs=pl.BlockSpec((_FIN_BLK, D), lambda i: (i, 0)),
        out_shape=jax.ShapeDtypeStruct((N, D), jnp.float32),
    )(a0, a1, deg_part, W, b2, noise)


def kernel(x, edge_index, W, b, noise):
    xn = _normalize(x)

    src = edge_index[0]
    dst = edge_index[1]
    pad = EPAD - E
    src_p = jnp.concatenate([src, jnp.zeros((pad,), jnp.int32)])
    dst_p = jnp.concatenate([dst, jnp.full((pad,), N, jnp.int32)])
    idx4 = jnp.stack(
        [src_p.reshape(NW, NCHUNK, CHUNK), dst_p.reshape(NW, NCHUNK, CHUNK)],
        axis=2,
    )

    zrow = jnp.zeros((ROWS_PER_TILE, D), jnp.float32)
    zdeg = jnp.zeros((NPAD,), jnp.float32)

    agg_part, deg_part = _sc_scatter(xn, idx4, zrow, zdeg)

    return _final(agg_part[0, :N], agg_part[1, :N], deg_part.T, W,
                  b.reshape(1, D), noise)
